# speedup vs baseline: 5.9732x; 5.9732x over previous
import functools

import jax
import jax.numpy as jnp
from jax import lax
from jax.experimental import pallas as pl
from jax.experimental.pallas import tpu as pltpu
from jax.experimental.pallas import tpu_sc as plsc

NUM_CORES = 2
NUM_SUBCORES = 16
NUM_WORKERS = NUM_CORES * NUM_SUBCORES
WIN = 128


def _round_up(v, m):
    return (v + m - 1) // m * m


def _sc_aggregate(x, row, col, n_pad, e_per_w):
    d = x.shape[1]
    num_win = e_per_w // WIN
    zrows = n_pad // NUM_SUBCORES
    mesh = plsc.VectorSubcoreMesh(core_axis_name="c", subcore_axis_name="s")

    def body(x_hbm, row_hbm, col_hbm, out_hbm, cidx, ridx, rows, acc, sem):
        c = lax.axis_index("c")
        s = lax.axis_index("s")
        w = s * NUM_CORES + c
        base = w * e_per_w

        @pl.loop(0, WIN)
        def _(r):
            @pl.loop(0, d // 16)
            def _(k):
                rows[r, pl.ds(k * 16, 16)] = jnp.zeros((16,), jnp.float32)

        @pl.loop(0, zrows // WIN)
        def _(b):
            pltpu.sync_copy(rows, acc.at[pl.ds(s * zrows + b * WIN, WIN)])

        plsc.subcore_barrier()

        @pl.loop(0, num_win)
        def _(t):
            off = base + t * WIN
            pltpu.sync_copy(row_hbm.at[pl.ds(off, WIN)], ridx)
            pltpu.sync_copy(col_hbm.at[pl.ds(off, WIN)], cidx)
            pltpu.async_copy(x_hbm.at[cidx], rows, sem).wait()
            pltpu.sync_copy(rows, acc.at[ridx], add=True)

        plsc.subcore_barrier()
        pltpu.sync_copy(acc.at[pl.ds(s * zrows, zrows)],
                        out_hbm.at[c, pl.ds(s * zrows, zrows)])

    kern = pl.kernel(
        body,
        out_type=jax.ShapeDtypeStruct((NUM_CORES, n_pad, d), jnp.float32),
        mesh=mesh,
        scratch_types=[
            pltpu.VMEM((WIN,), jnp.int32),
            pltpu.VMEM((WIN,), jnp.int32),
            pltpu.VMEM((WIN, d), jnp.float32),
            pltpu.VMEM_SHARED((n_pad, d), jnp.float32),
            pltpu.SemaphoreType.DMA,
        ],
    )
    return kern(x, row, col)


def _tc_combine(p, x, w_mat):
    n, d = x.shape
    br = 1000
    assert n % br == 0

    def body(p0_ref, p1_ref, x_ref, w_ref, o_ref):
        s = p0_ref[0] + p1_ref[0] + x_ref[...]
        o_ref[...] = jnp.dot(s, w_ref[...], preferred_element_type=jnp.float32)

    return pl.pallas_call(
        body,
        grid=(n // br,),
        in_specs=[
            pl.BlockSpec((1, br, d), lambda i: (0, i, 0)),
            pl.BlockSpec((1, br, d), lambda i: (1, i, 0)),
            pl.BlockSpec((br, d), lambda i: (i, 0)),
            pl.BlockSpec((d, d), lambda i: (0, 0)),
        ],
        out_specs=pl.BlockSpec((br, d), lambda i: (i, 0)),
        out_shape=jax.ShapeDtypeStruct((n, d), jnp.float32),
    )(p, p, x, w_mat)


@jax.jit
def kernel(x, edge_index, W):
    n, d = x.shape
    e = edge_index.shape[1]
    e_pad = _round_up(e, NUM_WORKERS * WIN)
    e_per_w = e_pad // NUM_WORKERS
    n_pad = _round_up(n + 1, NUM_SUBCORES * WIN)

    pad = e_pad - e
    row = jnp.concatenate(
        [edge_index[0], jnp.full((pad,), n, edge_index.dtype)])
    col = jnp.concatenate(
        [edge_index[1], jnp.zeros((pad,), edge_index.dtype)])

    p = _sc_aggregate(x, row, col, n_pad, e_per_w)
    return _tc_combine(p, x, W)
